# baseline (speedup 1.0000x reference)
import functools

import jax
import jax.numpy as jnp
from jax import lax
from jax.experimental import pallas as pl
from jax.experimental.pallas import tpu as pltpu
from jax.experimental.pallas import tpu_sc as plsc

VOCAB = 100000
D = 64
DP = 128
S = 200
B = 1024
NW = 32
SEQ_PER_W = B // NW
LANES = 16
NB = 3
SPLIT = 128
HALF_ROWS = VOCAB // 2


def _make_kernel():
    mesh = plsc.VectorSubcoreMesh(core_axis_name="c", subcore_axis_name="s")

    @functools.partial(
        pl.kernel,
        mesh=mesh,
        out_type=jax.ShapeDtypeStruct((B * S, D), jnp.float32),
        scratch_types=[
            pltpu.VMEM((SEQ_PER_W * S,), jnp.int32),
            pltpu.VMEM((NB, S, DP), jnp.float32),
            pltpu.VMEM((S, D), jnp.float32),
            pltpu.VMEM((S * D,), jnp.float32),
            pltpu.SemaphoreType.DMA((NB,)),
            pltpu.SemaphoreType.DMA,
        ],
    )
    def k(seq_hbm, tok_hbm, pos_hbm, out_hbm,
          idx_v, rows_v, stage_v, pos_v, gsem, osem):
        wid = lax.axis_index("s") * 2 + lax.axis_index("c")
        base_row = wid * (SEQ_PER_W * S)

        pltpu.sync_copy(seq_hbm.at[pl.ds(base_row, SEQ_PER_W * S)], idx_v)
        pltpu.sync_copy(pos_hbm, pos_v)

        def remap_body(i, c):
            sl = pl.ds(i * LANES, LANES)
            raw = idx_v[sl]
            idx_v[sl] = (lax.shift_right_logical(raw, 1)
                         + (raw & 1) * HALF_ROWS)
            return c
        lax.fori_loop(0, SEQ_PER_W * S // LANES, remap_body, 0)

        def gather_descs(s, b):
            off = s * S
            c1 = pltpu.make_async_copy(
                tok_hbm.at[idx_v.at[pl.ds(off, SPLIT)]],
                rows_v.at[b, pl.ds(0, SPLIT)], gsem.at[b])
            c2 = pltpu.make_async_copy(
                tok_hbm.at[idx_v.at[pl.ds(off + SPLIT, S - SPLIT)]],
                rows_v.at[b, pl.ds(SPLIT, S - SPLIT)], gsem.at[b])
            return c1, c2

        def out_desc(s):
            return pltpu.make_async_copy(
                stage_v, out_hbm.at[pl.ds(base_row + s * S, S)], osem)

        def add_slot(b):
            def add_body(i, c):
                r = i * 2
                for j in range(2):
                    for q in range(D // LANES):
                        stage_v[r + j, pl.ds(q * LANES, LANES)] = (
                            rows_v[b, r + j, pl.ds(q * LANES, LANES)]
                            + pos_v[pl.ds((r + j) * D + q * LANES, LANES)])
                return c
            lax.fori_loop(0, S // 2, add_body, 0)

        def step(s, b, first, last):
            c1, c2 = gather_descs(s, b)
            c1.wait()
            c2.wait()
            if not first:
                out_desc(s - 1).wait()
            add_slot(b)
            out_desc(s).start()
            want_prefetch = (not last and
                             (not isinstance(s, int) or s + NB - 1 < SEQ_PER_W))
            if want_prefetch:
                t = s + NB - 1
                bt = (b + NB - 1) % NB
                g1, g2 = gather_descs(t, bt)
                g1.start()
                g2.start()

        for s0 in range(NB - 1):
            c1, c2 = gather_descs(s0, s0)
            c1.start()
            c2.start()

        def body(i, carry):
            s_base = i * NB
            for b in range(NB):
                s = s_base + b
                step(s, b, first=False, last=False)
            return carry

        MAIN = (SEQ_PER_W - NB) // NB
        for s in range(NB):
            step(s, s % NB, first=(s == 0), last=False)
        lax.fori_loop(1, 1 + MAIN, body, 0)
        for s in range(NB * (1 + MAIN), SEQ_PER_W):
            step(s, s % NB, first=False, last=(s == SEQ_PER_W - 1))

        out_desc(SEQ_PER_W - 1).wait()

    return k


_kernel_call = _make_kernel()


def kernel(sequence, token_table, pos_table):
    seq_flat = sequence.reshape(-1).astype(jnp.int32)
    pos_flat = pos_table.reshape(-1)
    flat_p = jnp.pad(token_table.reshape(-1), (0, 2 * D))
    even = flat_p[:VOCAB * D].reshape(HALF_ROWS, DP)
    odd = flat_p[D:VOCAB * D + D].reshape(HALF_ROWS, DP)
    tok2 = jnp.concatenate([even, odd], axis=0)
    out = _kernel_call(seq_flat, tok2, pos_flat)
    return out.reshape(B, S, D)
